# Initial kernel scaffold; baseline (speedup 1.0000x reference)
import numpy as np
import jax
import jax.numpy as jnp
from jax.experimental import pallas as pl

_N = 100
_K = 31
_C = float(1.0 / np.sqrt(202.0))
_SELF_A = 0.5
_SELF_H = float(1.0 / 101.0)


def _encoder_block(obs_ref, W1_ref, b1_ref, W2_ref, b2_ref,
                   wsS_ref, wsN_ref, bs_ref, x1_ref):
    GB = obs_ref.shape[0]
    obs = obs_ref[...]
    o2 = obs.reshape(GB * _N, obs.shape[-1])
    xw1 = jnp.dot(o2, W1_ref[...], preferred_element_type=jnp.float32)
    b1 = b1_ref[...]
    s1 = jnp.sum(xw1.reshape(GB, _N, -1), axis=1)
    h1a = jnp.maximum(_SELF_A * xw1 + b1, 0.0)
    h1h = jnp.maximum(_C * s1 + b1, 0.0)

    W2 = W2_ref[...]
    xw2a = jnp.dot(h1a, W2, preferred_element_type=jnp.float32)
    xw2h = jnp.dot(h1h, W2, preferred_element_type=jnp.float32)
    s_h1a = jnp.sum(h1a.reshape(GB, _N, -1), axis=1)
    xw2s = jnp.dot(s_h1a, W2, preferred_element_type=jnp.float32)
    b2 = b2_ref[...]
    h2a = jnp.maximum(_C * xw2h[:, None, :] + _SELF_A * xw2a.reshape(GB, _N, -1)
                      + b2[None], 0.0)
    h2h = jnp.maximum(_C * xw2s + _SELF_H * xw2h + b2, 0.0)

    wsS = wsS_ref[...]
    wsN = wsN_ref[...]
    bs = bs_ref[0, 0]
    sa_self = jnp.sum(h2a * wsS[None], axis=-1)
    sh_nbr = jnp.sum(h2h * wsN, axis=-1)
    score_a = jnp.tanh(sa_self + sh_nbr[:, None] + bs)
    s_h2a = jnp.sum(h2a, axis=1)
    score_h = jnp.tanh(jnp.sum(h2h * wsS, axis=-1)
                       + jnp.sum(s_h2a * wsN, axis=-1) + bs)

    si = score_a[:, :, None]
    sj = score_a[:, None, :]
    ii = jax.lax.broadcasted_iota(jnp.int32, (GB, _N, _N), 1)
    jj = jax.lax.broadcasted_iota(jnp.int32, (GB, _N, _N), 2)
    beats = (sj > si) | ((sj == si) & (jj < ii))
    rank_a = jnp.sum(beats.astype(jnp.float32), axis=2)
    hub_beats = score_h[:, None] >= score_a
    rank_a = rank_a + hub_beats.astype(jnp.float32)
    rank_h = jnp.sum((score_a > score_h[:, None]).astype(jnp.float32), axis=1)
    mask_a = rank_a < _K
    mask_h = rank_h < _K

    wa = jnp.where(mask_a, score_a, 0.0)
    wh = jnp.where(mask_h, score_h, 0.0)
    mean_pool = (jnp.sum(wa[:, :, None] * h2a, axis=1)
                 + wh[:, None] * h2h) * (1.0 / _K)
    neg = jnp.float32(-jnp.inf)
    xpa = jnp.where(mask_a[:, :, None], score_a[:, :, None] * h2a, neg)
    xph = jnp.where(mask_h[:, None], score_h[:, None] * h2h, neg)
    max_pool = jnp.maximum(jnp.max(xpa, axis=1), xph)
    x1_ref[...] = jnp.concatenate([max_pool, mean_pool], axis=1)


def _head_block(x1_ref, Wl1_ref, bl1_ref, Wl2_ref, bl2_ref, out_ref):
    h = jnp.maximum(jnp.dot(x1_ref[...], Wl1_ref[...],
                            preferred_element_type=jnp.float32) + bl1_ref[...], 0.0)
    out_ref[...] = jnp.dot(h, Wl2_ref[...],
                           preferred_element_type=jnp.float32) + bl2_ref[...]


def kernel(obs, is_alive, W1, b1, W2, b2, Ws_self, Ws_nbr, bs, Wl1, bl1, Wl2, bl2):
    B, n, f = obs.shape
    H = W2.shape[1]
    GB = 8
    x1 = pl.pallas_call(
        _encoder_block,
        grid=(B // GB,),
        in_specs=[
            pl.BlockSpec((GB, n, f), lambda i: (i, 0, 0)),
            pl.BlockSpec(W1.shape, lambda i: (0, 0)),
            pl.BlockSpec((1, b1.shape[0]), lambda i: (0, 0)),
            pl.BlockSpec(W2.shape, lambda i: (0, 0)),
            pl.BlockSpec((1, b2.shape[0]), lambda i: (0, 0)),
            pl.BlockSpec((1, H), lambda i: (0, 0)),
            pl.BlockSpec((1, H), lambda i: (0, 0)),
            pl.BlockSpec((1, 1), lambda i: (0, 0)),
        ],
        out_specs=pl.BlockSpec((GB, 2 * H), lambda i: (i, 0)),
        out_shape=jax.ShapeDtypeStruct((B, 2 * H), jnp.float32),
    )(obs, W1, b1.reshape(1, -1), W2, b2.reshape(1, -1),
      Ws_self.reshape(1, -1), Ws_nbr.reshape(1, -1), bs.reshape(1, 1))

    MB = 128
    out = pl.pallas_call(
        _head_block,
        grid=(B // MB,),
        in_specs=[
            pl.BlockSpec((MB, 2 * H), lambda i: (i, 0)),
            pl.BlockSpec(Wl1.shape, lambda i: (0, 0)),
            pl.BlockSpec((1, H), lambda i: (0, 0)),
            pl.BlockSpec(Wl2.shape, lambda i: (0, 0)),
            pl.BlockSpec((1, H), lambda i: (0, 0)),
        ],
        out_specs=pl.BlockSpec((MB, H), lambda i: (i, 0)),
        out_shape=jax.ShapeDtypeStruct((B, H), jnp.float32),
    )(x1, Wl1, bl1.reshape(1, -1), Wl2, bl2.reshape(1, -1))
    return out


# speedup vs baseline: 6.4992x; 6.4992x over previous
import numpy as np
import jax
import jax.numpy as jnp
from jax.experimental import pallas as pl

_N = 100
_K = 31


def _bdot(a, b):
    return jnp.dot(a.astype(jnp.bfloat16), b.astype(jnp.bfloat16),
                   preferred_element_type=jnp.float32)


def _b16(a):
    return a.astype(jnp.bfloat16).astype(jnp.float32)


def _encoder_block(obs_ref, W1_ref, b1_ref, W2_ref, b2_ref,
                   wsS_ref, wsN_ref, bs_ref, x1_ref):
    GB = obs_ref.shape[0]
    dinv_a = jax.lax.rsqrt(jnp.float32(2.0))
    dinv_h = jax.lax.rsqrt(jnp.float32(101.0))
    C = dinv_h * dinv_a
    SA = dinv_a * dinv_a
    SH = dinv_h * dinv_h

    obs = obs_ref[...]
    o2 = obs.reshape(GB * _N, obs.shape[-1])
    xw1 = _bdot(o2, W1_ref[...])
    b1 = b1_ref[...]
    s1 = jnp.sum((C * xw1).reshape(GB, _N, -1), axis=1)
    h1a = jnp.maximum(SA * xw1 + b1, 0.0)
    h1h = jnp.maximum(s1 + b1, 0.0)

    W2 = W2_ref[...]
    xw2a = _bdot(h1a, W2)
    xw2h = _bdot(h1h, W2)
    b2 = b2_ref[...]
    s2 = jnp.sum((C * xw2a).reshape(GB, _N, -1), axis=1)
    h2a = jnp.maximum(C * xw2h[:, None, :]
                      + SA * xw2a.reshape(GB, _N, -1) + b2[None], 0.0)
    h2h = jnp.maximum(s2 + SH * xw2h + b2, 0.0)

    wsS = _b16(wsS_ref[...])
    wsN = _b16(wsN_ref[...])
    bs = bs_ref[0, 0]
    h2a_r = _b16(h2a)
    h2h_r = _b16(h2h)
    sa_self = jnp.sum(h2a_r * wsS[None], axis=-1)
    sh_nbr = jnp.sum(h2h_r * wsN, axis=-1)
    score_a = jnp.tanh(sa_self + sh_nbr[:, None] + bs)
    nbr_h = _b16(jnp.sum(h2a, axis=1))
    score_h = jnp.tanh(jnp.sum(h2h_r * wsS, axis=-1)
                       + jnp.sum(nbr_h * wsN, axis=-1) + bs)

    si = score_a[:, :, None]
    sj = score_a[:, None, :]
    ii = jax.lax.broadcasted_iota(jnp.int32, (GB, _N, _N), 1)
    jj = jax.lax.broadcasted_iota(jnp.int32, (GB, _N, _N), 2)
    beats = (sj > si) | ((sj == si) & (jj < ii))
    rank_a = jnp.sum(beats.astype(jnp.float32), axis=2)
    hub_beats = score_h[:, None] >= score_a
    rank_a = rank_a + hub_beats.astype(jnp.float32)
    rank_h = jnp.sum((score_a > score_h[:, None]).astype(jnp.float32), axis=1)
    mask_a = rank_a < _K
    mask_h = rank_h < _K

    wa = jnp.where(mask_a, score_a, 0.0)
    wh = jnp.where(mask_h, score_h, 0.0)
    mean_pool = (jnp.sum(wa[:, :, None] * h2a, axis=1)
                 + wh[:, None] * h2h) * jnp.float32(1.0 / _K)
    neg = jnp.float32(-jnp.inf)
    xpa = jnp.where(mask_a[:, :, None], score_a[:, :, None] * h2a, neg)
    xph = jnp.where(mask_h[:, None], score_h[:, None] * h2h, neg)
    max_pool = jnp.maximum(jnp.max(xpa, axis=1), xph)
    x1_ref[...] = jnp.concatenate([max_pool, mean_pool], axis=1)


def _head_block(x1_ref, Wl1_ref, bl1_ref, Wl2_ref, bl2_ref, out_ref):
    h = jnp.maximum(_bdot(x1_ref[...], Wl1_ref[...]) + bl1_ref[...], 0.0)
    out_ref[...] = _bdot(h, Wl2_ref[...]) + bl2_ref[...]


def kernel(obs, is_alive, W1, b1, W2, b2, Ws_self, Ws_nbr, bs, Wl1, bl1, Wl2, bl2):
    B, n, f = obs.shape
    H = W2.shape[1]
    GB = 8
    x1 = pl.pallas_call(
        _encoder_block,
        grid=(B // GB,),
        in_specs=[
            pl.BlockSpec((GB, n, f), lambda i: (i, 0, 0)),
            pl.BlockSpec(W1.shape, lambda i: (0, 0)),
            pl.BlockSpec((1, b1.shape[0]), lambda i: (0, 0)),
            pl.BlockSpec(W2.shape, lambda i: (0, 0)),
            pl.BlockSpec((1, b2.shape[0]), lambda i: (0, 0)),
            pl.BlockSpec((1, H), lambda i: (0, 0)),
            pl.BlockSpec((1, H), lambda i: (0, 0)),
            pl.BlockSpec((1, 1), lambda i: (0, 0)),
        ],
        out_specs=pl.BlockSpec((GB, 2 * H), lambda i: (i, 0)),
        out_shape=jax.ShapeDtypeStruct((B, 2 * H), jnp.float32),
    )(obs, W1, b1.reshape(1, -1), W2, b2.reshape(1, -1),
      Ws_self.reshape(1, -1), Ws_nbr.reshape(1, -1), bs.reshape(1, 1))

    MB = 128 if B % 128 == 0 else B
    out = pl.pallas_call(
        _head_block,
        grid=(B // MB,),
        in_specs=[
            pl.BlockSpec((MB, 2 * H), lambda i: (i, 0)),
            pl.BlockSpec(Wl1.shape, lambda i: (0, 0)),
            pl.BlockSpec((1, H), lambda i: (0, 0)),
            pl.BlockSpec(Wl2.shape, lambda i: (0, 0)),
            pl.BlockSpec((1, H), lambda i: (0, 0)),
        ],
        out_specs=pl.BlockSpec((MB, H), lambda i: (i, 0)),
        out_shape=jax.ShapeDtypeStruct((B, H), jnp.float32),
    )(x1, Wl1, bl1.reshape(1, -1), Wl2, bl2.reshape(1, -1))
    return out


# speedup vs baseline: 31.9884x; 4.9219x over previous
import numpy as np
import jax
import jax.numpy as jnp
from jax.experimental import pallas as pl

_N = 100
_K = 31


def _bdot(a, b):
    return jnp.dot(a.astype(jnp.bfloat16), b.astype(jnp.bfloat16),
                   preferred_element_type=jnp.float32)


def _b16(a):
    return a.astype(jnp.bfloat16).astype(jnp.float32)


def _encoder_block(obs_ref, W1_ref, b1_ref, W2_ref, b2_ref,
                   wsS_ref, wsN_ref, bs_ref, tri_ref, x1_ref):
    GB = obs_ref.shape[0]
    dinv_a = jax.lax.rsqrt(jnp.float32(2.0))
    dinv_h = jax.lax.rsqrt(jnp.float32(101.0))
    C = dinv_h * dinv_a
    SA = dinv_a * dinv_a
    SH = dinv_h * dinv_h

    obs = obs_ref[...]
    o2 = obs.reshape(GB * _N, obs.shape[-1])
    xw1 = _bdot(o2, W1_ref[...])
    b1 = b1_ref[...]
    s1 = jnp.sum((C * xw1).reshape(GB, _N, -1), axis=1)
    h1a = jnp.maximum(SA * xw1 + b1, 0.0)
    h1h = jnp.maximum(s1 + b1, 0.0)

    W2 = W2_ref[...]
    xw2a = _bdot(h1a, W2)
    xw2h = _bdot(h1h, W2)
    b2 = b2_ref[...]
    s2 = jnp.sum((C * xw2a).reshape(GB, _N, -1), axis=1)
    h2a = jnp.maximum(C * xw2h[:, None, :]
                      + SA * xw2a.reshape(GB, _N, -1) + b2[None], 0.0)
    h2h = jnp.maximum(s2 + SH * xw2h + b2, 0.0)

    wsS = wsS_ref[...]
    wsN = wsN_ref[...]
    bs = bs_ref[0, 0]
    sa_self = _bdot(h2a.reshape(GB * _N, -1), wsS)
    sh_nbr = _bdot(h2h, wsN)
    nbr_h = jnp.sum(h2a, axis=1)
    score_h = jnp.tanh(_bdot(h2h, wsS) + _bdot(nbr_h, wsN) + bs)
    score_a3 = jnp.tanh(sa_self.reshape(GB, _N, 1)
                        + sh_nbr[:, None, :] + bs)
    scores3 = jnp.concatenate([score_h[:, None, :], score_a3], axis=1)

    sj3 = jnp.swapaxes(scores3, 1, 2)
    tri = tri_ref[...] != 0.0
    beats = (sj3 > scores3) | ((sj3 == scores3) & tri[None])
    rank3 = jnp.sum(beats.astype(jnp.float32), axis=2, keepdims=True)
    mask3 = rank3 < _K

    h2_all = jnp.concatenate([h2h[:, None, :], h2a], axis=1)
    w3 = jnp.where(mask3, scores3, 0.0)
    mean_pool = jnp.sum(w3 * h2_all, axis=1) * jnp.float32(1.0 / _K)
    xp = jnp.where(mask3, scores3 * h2_all, jnp.float32(-jnp.inf))
    max_pool = jnp.max(xp, axis=1)
    x1_ref[...] = jnp.concatenate([max_pool, mean_pool], axis=1)


def _head_block(x1_ref, Wl1_ref, bl1_ref, Wl2_ref, bl2_ref, out_ref):
    h = jnp.maximum(_bdot(x1_ref[...], Wl1_ref[...]) + bl1_ref[...], 0.0)
    out_ref[...] = _bdot(h, Wl2_ref[...]) + bl2_ref[...]


def kernel(obs, is_alive, W1, b1, W2, b2, Ws_self, Ws_nbr, bs, Wl1, bl1, Wl2, bl2):
    B, n, f = obs.shape
    H = W2.shape[1]
    npg = n + 1
    GB = 8
    tri = jnp.asarray(np.tril(np.ones((npg, npg), np.float32), k=-1))
    x1 = pl.pallas_call(
        _encoder_block,
        grid=(B // GB,),
        in_specs=[
            pl.BlockSpec((GB, n, f), lambda i: (i, 0, 0)),
            pl.BlockSpec(W1.shape, lambda i: (0, 0)),
            pl.BlockSpec((1, b1.shape[0]), lambda i: (0, 0)),
            pl.BlockSpec(W2.shape, lambda i: (0, 0)),
            pl.BlockSpec((1, b2.shape[0]), lambda i: (0, 0)),
            pl.BlockSpec((H, 1), lambda i: (0, 0)),
            pl.BlockSpec((H, 1), lambda i: (0, 0)),
            pl.BlockSpec((1, 1), lambda i: (0, 0)),
            pl.BlockSpec((npg, npg), lambda i: (0, 0)),
        ],
        out_specs=pl.BlockSpec((GB, 2 * H), lambda i: (i, 0)),
        out_shape=jax.ShapeDtypeStruct((B, 2 * H), jnp.float32),
    )(obs, W1, b1.reshape(1, -1), W2, b2.reshape(1, -1),
      Ws_self, Ws_nbr, bs.reshape(1, 1), tri)

    MB = 128 if B % 128 == 0 else B
    out = pl.pallas_call(
        _head_block,
        grid=(B // MB,),
        in_specs=[
            pl.BlockSpec((MB, 2 * H), lambda i: (i, 0)),
            pl.BlockSpec(Wl1.shape, lambda i: (0, 0)),
            pl.BlockSpec((1, H), lambda i: (0, 0)),
            pl.BlockSpec(Wl2.shape, lambda i: (0, 0)),
            pl.BlockSpec((1, H), lambda i: (0, 0)),
        ],
        out_specs=pl.BlockSpec((MB, H), lambda i: (i, 0)),
        out_shape=jax.ShapeDtypeStruct((B, H), jnp.float32),
    )(x1, Wl1, bl1.reshape(1, -1), Wl2, bl2.reshape(1, -1))
    return out
